# baseline (speedup 1.0000x reference)
import functools

import jax
import jax.numpy as jnp
from jax import lax
from jax.experimental import pallas as pl
from jax.experimental.pallas import tpu as pltpu
from jax.experimental.pallas import tpu_sc as plsc

N_DATA = 10000
N_NODES = 10128
N_PAD = 10240
ROWS_PT = 640
H = 16
D_FEAT = 128
E_TOTAL = 320000
L_WIN = 2000
NWIN = 5
NC = 2
NS = 16

_MESH = plsc.VectorSubcoreMesh(core_axis_name="c", subcore_axis_name="s")
_SC_PARAMS = pltpu.CompilerParams(use_tc_tiling_on_sc=False)


@functools.partial(
    pl.kernel,
    out_type=jax.ShapeDtypeStruct((NC, N_PAD), jnp.float32),
    mesh=_MESH,
    compiler_params=_SC_PARAMS,
    scratch_types=[
        pltpu.VMEM((NWIN, L_WIN), jnp.int32),
        pltpu.VMEM((L_WIN,), jnp.float32),
        pltpu.VMEM_SHARED((N_PAD,), jnp.float32),
        pltpu.SemaphoreType.DMA,
    ],
)
def _deg_kernel(ei_hbm, ones_hbm, zeros_hbm, out_hbm, idx_v, ones_v, deg_sh, dsem):
    sid = lax.axis_index("s")
    cid = lax.axis_index("c")
    wid = cid * NS + sid
    row_lo = sid * ROWS_PT
    pltpu.sync_copy(zeros_hbm, deg_sh.at[pl.ds(row_lo, ROWS_PT)])
    pltpu.sync_copy(ones_hbm, ones_v)
    pltpu.sync_copy(ei_hbm.at[1, pl.ds(wid * NWIN, NWIN)], idx_v)
    plsc.subcore_barrier()

    def body(w, carry):
        pltpu.async_copy(ones_v, deg_sh.at[idx_v.at[w]], dsem, add=True)
        return carry

    lax.fori_loop(0, NWIN, body, 0)

    def drain(w, carry):
        pltpu.make_async_copy(ones_v, deg_sh.at[idx_v.at[w]], dsem).wait()
        return carry

    lax.fori_loop(0, NWIN, drain, 0)
    plsc.subcore_barrier()
    pltpu.sync_copy(deg_sh.at[pl.ds(row_lo, ROWS_PT)],
                    out_hbm.at[cid, pl.ds(row_lo, ROWS_PT)])


@functools.partial(
    pl.kernel,
    out_type=jax.ShapeDtypeStruct((NC, N_PAD, H), jnp.float32),
    mesh=_MESH,
    compiler_params=_SC_PARAMS,
    scratch_types=[
        pltpu.VMEM((NWIN, L_WIN), jnp.int32),
        pltpu.VMEM((NWIN, L_WIN), jnp.int32),
        pltpu.VMEM((2, L_WIN, H), jnp.float32),
        pltpu.VMEM_SHARED((N_PAD, H), jnp.float32),
        pltpu.VMEM_SHARED((N_PAD, H), jnp.float32),
        pltpu.SemaphoreType.DMA((2,)),
        pltpu.SemaphoreType.DMA((2,)),
    ],
)
def _conv_kernel(g_hbm, ei_hbm, out_hbm,
                 src_v, dst_v, rows_v, acc_sh, g_sh, gsem, ssem):
    sid = lax.axis_index("s")
    cid = lax.axis_index("c")
    wid = cid * NS + sid
    row_lo = sid * ROWS_PT
    pltpu.sync_copy(g_hbm.at[pl.ds(row_lo, ROWS_PT)],
                    acc_sh.at[pl.ds(row_lo, ROWS_PT)])
    pltpu.sync_copy(g_hbm.at[pl.ds(row_lo, ROWS_PT)],
                    g_sh.at[pl.ds(row_lo, ROWS_PT)])
    pltpu.sync_copy(ei_hbm.at[0, pl.ds(wid * NWIN, NWIN)], src_v)
    pltpu.sync_copy(ei_hbm.at[1, pl.ds(wid * NWIN, NWIN)], dst_v)
    plsc.subcore_barrier()

    pltpu.async_copy(g_sh.at[src_v.at[0]], rows_v.at[0], gsem.at[0])

    def body(w, carry):
        @pl.when(jnp.logical_and(w >= 1, w + 1 < NWIN))
        def _():
            pltpu.make_async_copy(rows_v.at[(w + 1) % 2],
                                  acc_sh.at[dst_v.at[w - 1]],
                                  ssem.at[(w + 1) % 2]).wait()

        @pl.when(w + 1 < NWIN)
        def _():
            pltpu.async_copy(g_sh.at[src_v.at[w + 1]],
                             rows_v.at[(w + 1) % 2], gsem.at[(w + 1) % 2])

        pltpu.make_async_copy(g_sh.at[src_v.at[w]],
                              rows_v.at[w % 2], gsem.at[w % 2]).wait()
        pltpu.async_copy(rows_v.at[w % 2], acc_sh.at[dst_v.at[w]],
                         ssem.at[w % 2], add=True)
        return carry

    lax.fori_loop(0, NWIN, body, 0)
    pltpu.make_async_copy(rows_v.at[(NWIN - 2) % 2],
                          acc_sh.at[dst_v.at[NWIN - 2]],
                          ssem.at[(NWIN - 2) % 2]).wait()
    pltpu.make_async_copy(rows_v.at[(NWIN - 1) % 2],
                          acc_sh.at[dst_v.at[NWIN - 1]],
                          ssem.at[(NWIN - 1) % 2]).wait()
    plsc.subcore_barrier()
    pltpu.sync_copy(acc_sh.at[pl.ds(row_lo, ROWS_PT)],
                    out_hbm.at[cid, pl.ds(row_lo, ROWS_PT)])


def _mm_body(h_ref, w_ref, o_ref):
    o_ref[pl.ds(0, N_NODES), :] = jnp.dot(h_ref[...], w_ref[...],
                                          preferred_element_type=jnp.float32)
    o_ref[pl.ds(N_NODES, N_PAD - N_NODES), :] = jnp.zeros(
        (N_PAD - N_NODES, H), jnp.float32)


def _glue1_body(pd_ref, hw_ref, dis_ref, g_ref):
    deg = pd_ref[0] + pd_ref[1] + 1.0
    rows = lax.broadcasted_iota(jnp.int32, (N_PAD, 1), 0)
    dism = jnp.where(rows < N_NODES, lax.rsqrt(deg), 0.0)
    dis_ref[...] = dism
    g_ref[...] = hw_ref[...] * dism


def _glue2_body(p_ref, g1_ref, dis_ref, b_ref, w_ref, g2_ref):
    acc = p_ref[0] + p_ref[1] - g1_ref[...]
    h1 = jnp.maximum(acc * dis_ref[...] + b_ref[...], 0.0)
    g2_ref[...] = jnp.dot(h1, w_ref[...],
                          preferred_element_type=jnp.float32) * dis_ref[...]


def _final_body(p_ref, g2_ref, dis_ref, bg2_ref, h0_ref,
                w1a_ref, b1a_ref, w1b_ref, b1b_ref,
                w2t_ref, w2c_ref, b2a_ref, w2b_ref, b2b_ref,
                out1_ref, out2_ref, si_ref, ei_ref, nn_ref, nf_ref,
                h2_scr):
    h2 = jnp.maximum((p_ref[0] + p_ref[1] - g2_ref[...]) * dis_ref[...]
                     + bg2_ref[...], 0.0)
    h2_scr[...] = h2
    t = jnp.maximum(jnp.dot(h2, w1a_ref[...],
                            preferred_element_type=jnp.float32)
                    + b1a_ref[...], 0.0)
    out1 = jnp.dot(t, w1b_ref[...],
                   preferred_element_type=jnp.float32) + b1b_ref[...]
    out1_ref[...] = out1[:N_NODES]

    rows = lax.broadcasted_iota(jnp.int32, (N_PAD, 1), 0)
    neg = jnp.float32(-1e30)
    m1 = jnp.where(rows < N_DATA, out1, neg)
    sidx = jnp.min(jnp.where(m1 == jnp.max(m1), rows, N_PAD))
    si_ref[0, 0] = sidx

    sf = h2_scr[pl.ds(sidx, 1), :]
    c2 = jnp.dot(sf, w2c_ref[...],
                 preferred_element_type=jnp.float32) + b2a_ref[...]
    t2 = jnp.maximum(jnp.dot(h2, w2t_ref[...],
                             preferred_element_type=jnp.float32) + c2, 0.0)
    out2 = jnp.dot(t2, w2b_ref[...],
                   preferred_element_type=jnp.float32) + b2b_ref[...]
    out2_ref[...] = out2[:N_NODES]

    bad = jnp.logical_or(rows == sidx, rows >= N_NODES)
    m2 = jnp.where(bad, neg, out2)
    eidx = jnp.min(jnp.where(m2 == jnp.max(m2), rows, N_PAD))
    ei_ref[0, 0] = eidx
    nn_ref[0, 0] = (eidx >= N_DATA).astype(jnp.int32)
    nf_ref[...] = h0_ref[pl.ds(eidx, 1), :]


def kernel(x, edge_index, candidates, W_g1, b_g1, W_g2, b_g2,
           W_m1a, b_m1a, W_m1b, b_m1b, W_m2a, b_m2a, W_m2b, b_m2b):
    h0 = jnp.concatenate([x, candidates], axis=0)
    eir = edge_index.astype(jnp.int32).reshape(2, 32 * NWIN, L_WIN)
    ones_w = jnp.ones((L_WIN,), jnp.float32)
    zeros_w = jnp.zeros((ROWS_PT,), jnp.float32)

    pdeg = _deg_kernel(eir, ones_w, zeros_w).reshape(NC, N_PAD, 1)

    hW1 = pl.pallas_call(
        _mm_body,
        out_shape=jax.ShapeDtypeStruct((N_PAD, H), jnp.float32),
    )(h0, W_g1)

    dism, g1 = pl.pallas_call(
        _glue1_body,
        out_shape=[jax.ShapeDtypeStruct((N_PAD, 1), jnp.float32),
                   jax.ShapeDtypeStruct((N_PAD, H), jnp.float32)],
    )(pdeg, hW1)

    p1 = _conv_kernel(g1, eir)

    g2 = pl.pallas_call(
        _glue2_body,
        out_shape=jax.ShapeDtypeStruct((N_PAD, H), jnp.float32),
    )(p1, g1, dism, b_g1.reshape(1, H), W_g2)

    p2 = _conv_kernel(g2, eir)

    out1p, out2p, si, ei, nn, nf = pl.pallas_call(
        _final_body,
        out_shape=[jax.ShapeDtypeStruct((N_NODES, 1), jnp.float32),
                   jax.ShapeDtypeStruct((N_NODES, 1), jnp.float32),
                   jax.ShapeDtypeStruct((1, 1), jnp.int32),
                   jax.ShapeDtypeStruct((1, 1), jnp.int32),
                   jax.ShapeDtypeStruct((1, 1), jnp.int32),
                   jax.ShapeDtypeStruct((1, D_FEAT), jnp.float32)],
        out_specs=[pl.BlockSpec(memory_space=pltpu.VMEM),
                   pl.BlockSpec(memory_space=pltpu.VMEM),
                   pl.BlockSpec(memory_space=pltpu.SMEM),
                   pl.BlockSpec(memory_space=pltpu.SMEM),
                   pl.BlockSpec(memory_space=pltpu.SMEM),
                   pl.BlockSpec(memory_space=pltpu.VMEM)],
        scratch_shapes=[pltpu.VMEM((N_PAD, H), jnp.float32)],
    )(p2, g2, dism, b_g2.reshape(1, H), h0,
      W_m1a, b_m1a.reshape(1, -1), W_m1b, b_m1b.reshape(1, 1),
      W_m2a[:H], W_m2a[H:], b_m2a.reshape(1, -1), W_m2b, b_m2b.reshape(1, 1))

    starting_idx = si.reshape(())
    ending_idx = ei.reshape(())
    is_new_node = nn.reshape(()).astype(jnp.bool_)
    new_features = nf.reshape(D_FEAT)
    return (starting_idx, ending_idx, is_new_node, new_features,
            out1p, out2p)


# speedup vs baseline: 1.0002x; 1.0002x over previous
import functools

import jax
import jax.numpy as jnp
from jax import lax
from jax.experimental import pallas as pl
from jax.experimental.pallas import tpu as pltpu
from jax.experimental.pallas import tpu_sc as plsc

N_DATA = 10000
N_NODES = 10128
N_PAD = 10240
ROWS_PT = 640
H = 16
D_FEAT = 128
E_TOTAL = 320000
L_WIN = 2000
NWIN = 5
NC = 2
NS = 16

_MESH = plsc.VectorSubcoreMesh(core_axis_name="c", subcore_axis_name="s")
_SC_PARAMS = pltpu.CompilerParams(use_tc_tiling_on_sc=False)


@functools.partial(
    pl.kernel,
    out_type=jax.ShapeDtypeStruct((NC, N_PAD), jnp.float32),
    mesh=_MESH,
    compiler_params=_SC_PARAMS,
    scratch_types=[
        pltpu.VMEM((NWIN, L_WIN), jnp.int32),
        pltpu.VMEM((L_WIN,), jnp.float32),
        pltpu.VMEM_SHARED((N_PAD,), jnp.float32),
        pltpu.SemaphoreType.DMA,
    ],
)
def _deg_kernel(ei_hbm, ones_hbm, zeros_hbm, out_hbm, idx_v, ones_v, deg_sh, dsem):
    sid = lax.axis_index("s")
    cid = lax.axis_index("c")
    wid = cid * NS + sid
    row_lo = sid * ROWS_PT
    pltpu.sync_copy(zeros_hbm, deg_sh.at[pl.ds(row_lo, ROWS_PT)])
    pltpu.sync_copy(ones_hbm, ones_v)
    pltpu.sync_copy(ei_hbm.at[1, pl.ds(wid * NWIN, NWIN)], idx_v)
    plsc.subcore_barrier()

    def body(w, carry):
        pltpu.async_copy(ones_v, deg_sh.at[idx_v.at[w]], dsem, add=True)
        return carry

    lax.fori_loop(0, NWIN, body, 0)

    def drain(w, carry):
        pltpu.make_async_copy(ones_v, deg_sh.at[idx_v.at[w]], dsem).wait()
        return carry

    lax.fori_loop(0, NWIN, drain, 0)
    plsc.subcore_barrier()
    pltpu.sync_copy(deg_sh.at[pl.ds(row_lo, ROWS_PT)],
                    out_hbm.at[cid, pl.ds(row_lo, ROWS_PT)])


@functools.partial(
    pl.kernel,
    out_type=jax.ShapeDtypeStruct((NC, N_PAD, H), jnp.float32),
    mesh=_MESH,
    compiler_params=_SC_PARAMS,
    scratch_types=[
        pltpu.VMEM((NWIN, L_WIN), jnp.int32),
        pltpu.VMEM((NWIN, L_WIN), jnp.int32),
        pltpu.VMEM((2, L_WIN, H), jnp.float32),
        pltpu.VMEM_SHARED((N_PAD, H), jnp.float32),
        pltpu.VMEM_SHARED((N_PAD, H), jnp.float32),
        pltpu.SemaphoreType.DMA((2,)),
    ],
)
def _conv_kernel(g_hbm, ei_hbm, out_hbm,
                 src_v, dst_v, rows_v, acc_sh, g_sh, gsem):
    sid = lax.axis_index("s")
    cid = lax.axis_index("c")
    wid = cid * NS + sid
    row_lo = sid * ROWS_PT
    pltpu.sync_copy(g_hbm.at[pl.ds(row_lo, ROWS_PT)],
                    acc_sh.at[pl.ds(row_lo, ROWS_PT)])
    pltpu.sync_copy(g_hbm.at[pl.ds(row_lo, ROWS_PT)],
                    g_sh.at[pl.ds(row_lo, ROWS_PT)])
    pltpu.sync_copy(ei_hbm.at[0, pl.ds(wid * NWIN, NWIN)], src_v)
    pltpu.sync_copy(ei_hbm.at[1, pl.ds(wid * NWIN, NWIN)], dst_v)
    plsc.subcore_barrier()

    pltpu.async_copy(g_sh.at[src_v.at[0]], rows_v.at[0], gsem.at[0])

    def body(w, carry):
        @pl.when(w + 1 < NWIN)
        def _():
            pltpu.async_copy(g_sh.at[src_v.at[w + 1]],
                             rows_v.at[(w + 1) % 2], gsem.at[(w + 1) % 2])

        pltpu.make_async_copy(g_sh.at[src_v.at[w]],
                              rows_v.at[w % 2], gsem.at[w % 2]).wait()
        pltpu.sync_copy(rows_v.at[w % 2], acc_sh.at[dst_v.at[w]], add=True)
        return carry

    lax.fori_loop(0, NWIN, body, 0)
    plsc.subcore_barrier()
    pltpu.sync_copy(acc_sh.at[pl.ds(row_lo, ROWS_PT)],
                    out_hbm.at[cid, pl.ds(row_lo, ROWS_PT)])


def _mm_body(h_ref, w_ref, o_ref):
    o_ref[pl.ds(0, N_NODES), :] = jnp.dot(h_ref[...], w_ref[...],
                                          preferred_element_type=jnp.float32)
    o_ref[pl.ds(N_NODES, N_PAD - N_NODES), :] = jnp.zeros(
        (N_PAD - N_NODES, H), jnp.float32)


def _glue1_body(pd_ref, hw_ref, dis_ref, g_ref):
    deg = pd_ref[0] + pd_ref[1] + 1.0
    rows = lax.broadcasted_iota(jnp.int32, (N_PAD, 1), 0)
    dism = jnp.where(rows < N_NODES, lax.rsqrt(deg), 0.0)
    dis_ref[...] = dism
    g_ref[...] = hw_ref[...] * dism


def _glue2_body(p_ref, g1_ref, dis_ref, b_ref, w_ref, g2_ref):
    acc = p_ref[0] + p_ref[1] - g1_ref[...]
    h1 = jnp.maximum(acc * dis_ref[...] + b_ref[...], 0.0)
    g2_ref[...] = jnp.dot(h1, w_ref[...],
                          preferred_element_type=jnp.float32) * dis_ref[...]


def _final_body(p_ref, g2_ref, dis_ref, bg2_ref, h0_ref,
                w1a_ref, b1a_ref, w1b_ref, b1b_ref,
                w2t_ref, w2c_ref, b2a_ref, w2b_ref, b2b_ref,
                out1_ref, out2_ref, si_ref, ei_ref, nn_ref, nf_ref,
                h2_scr):
    h2 = jnp.maximum((p_ref[0] + p_ref[1] - g2_ref[...]) * dis_ref[...]
                     + bg2_ref[...], 0.0)
    h2_scr[...] = h2
    t = jnp.maximum(jnp.dot(h2, w1a_ref[...],
                            preferred_element_type=jnp.float32)
                    + b1a_ref[...], 0.0)
    out1 = jnp.dot(t, w1b_ref[...],
                   preferred_element_type=jnp.float32) + b1b_ref[...]
    out1_ref[...] = out1[:N_NODES]

    rows = lax.broadcasted_iota(jnp.int32, (N_PAD, 1), 0)
    neg = jnp.float32(-1e30)
    m1 = jnp.where(rows < N_DATA, out1, neg)
    sidx = jnp.min(jnp.where(m1 == jnp.max(m1), rows, N_PAD))
    si_ref[0, 0] = sidx

    sf = h2_scr[pl.ds(sidx, 1), :]
    c2 = jnp.dot(sf, w2c_ref[...],
                 preferred_element_type=jnp.float32) + b2a_ref[...]
    t2 = jnp.maximum(jnp.dot(h2, w2t_ref[...],
                             preferred_element_type=jnp.float32) + c2, 0.0)
    out2 = jnp.dot(t2, w2b_ref[...],
                   preferred_element_type=jnp.float32) + b2b_ref[...]
    out2_ref[...] = out2[:N_NODES]

    bad = jnp.logical_or(rows == sidx, rows >= N_NODES)
    m2 = jnp.where(bad, neg, out2)
    eidx = jnp.min(jnp.where(m2 == jnp.max(m2), rows, N_PAD))
    ei_ref[0, 0] = eidx
    nn_ref[0, 0] = (eidx >= N_DATA).astype(jnp.int32)
    nf_ref[...] = h0_ref[pl.ds(eidx, 1), :]


def kernel(x, edge_index, candidates, W_g1, b_g1, W_g2, b_g2,
           W_m1a, b_m1a, W_m1b, b_m1b, W_m2a, b_m2a, W_m2b, b_m2b):
    h0 = jnp.concatenate([x, candidates], axis=0)
    eir = edge_index.astype(jnp.int32).reshape(2, 32 * NWIN, L_WIN)
    ones_w = jnp.ones((L_WIN,), jnp.float32)
    zeros_w = jnp.zeros((ROWS_PT,), jnp.float32)

    pdeg = _deg_kernel(eir, ones_w, zeros_w).reshape(NC, N_PAD, 1)

    hW1 = pl.pallas_call(
        _mm_body,
        out_shape=jax.ShapeDtypeStruct((N_PAD, H), jnp.float32),
    )(h0, W_g1)

    dism, g1 = pl.pallas_call(
        _glue1_body,
        out_shape=[jax.ShapeDtypeStruct((N_PAD, 1), jnp.float32),
                   jax.ShapeDtypeStruct((N_PAD, H), jnp.float32)],
    )(pdeg, hW1)

    p1 = _conv_kernel(g1, eir)

    g2 = pl.pallas_call(
        _glue2_body,
        out_shape=jax.ShapeDtypeStruct((N_PAD, H), jnp.float32),
    )(p1, g1, dism, b_g1.reshape(1, H), W_g2)

    p2 = _conv_kernel(g2, eir)

    out1p, out2p, si, ei, nn, nf = pl.pallas_call(
        _final_body,
        out_shape=[jax.ShapeDtypeStruct((N_NODES, 1), jnp.float32),
                   jax.ShapeDtypeStruct((N_NODES, 1), jnp.float32),
                   jax.ShapeDtypeStruct((1, 1), jnp.int32),
                   jax.ShapeDtypeStruct((1, 1), jnp.int32),
                   jax.ShapeDtypeStruct((1, 1), jnp.int32),
                   jax.ShapeDtypeStruct((1, D_FEAT), jnp.float32)],
        out_specs=[pl.BlockSpec(memory_space=pltpu.VMEM),
                   pl.BlockSpec(memory_space=pltpu.VMEM),
                   pl.BlockSpec(memory_space=pltpu.SMEM),
                   pl.BlockSpec(memory_space=pltpu.SMEM),
                   pl.BlockSpec(memory_space=pltpu.SMEM),
                   pl.BlockSpec(memory_space=pltpu.VMEM)],
        scratch_shapes=[pltpu.VMEM((N_PAD, H), jnp.float32)],
    )(p2, g2, dism, b_g2.reshape(1, H), h0,
      W_m1a, b_m1a.reshape(1, -1), W_m1b, b_m1b.reshape(1, 1),
      W_m2a[:H], W_m2a[H:], b_m2a.reshape(1, -1), W_m2b, b_m2b.reshape(1, 1))

    starting_idx = si.reshape(())
    ending_idx = ei.reshape(())
    is_new_node = nn.reshape(()).astype(jnp.bool_)
    new_features = nf.reshape(D_FEAT)
    return (starting_idx, ending_idx, is_new_node, new_features,
            out1p, out2p)
